# baseline (speedup 1.0000x reference)
import functools

import jax
import jax.numpy as jnp
from jax import lax
from jax.experimental import pallas as pl
from jax.experimental.pallas import tpu as pltpu
from jax.experimental.pallas import tpu_sc as plsc

D = 128
NC, NS = 2, 16
NW = NC * NS
CH = 128
HCH = 64
NBUF = 10


@functools.partial(jax.jit, static_argnames=("batch", "hist"))
def _lookup(idx3, table, *, batch, hist):
    mesh = plsc.VectorSubcoreMesh(core_axis_name="c", subcore_axis_name="s")

    @functools.partial(
        pl.kernel,
        out_type=jax.ShapeDtypeStruct((hist, batch, D), jnp.float32),
        mesh=mesh,
        scratch_types=[
            pltpu.VMEM((hist, CH), jnp.int32),
            pltpu.VMEM((NBUF, HCH, D), jnp.float32),
            pltpu.SemaphoreType.DMA((NBUF,)),
            pltpu.SemaphoreType.DMA((NBUF,)),
        ],
    )
    def body(table_hbm, idx_hbm, out_hbm, idx_v, rows_v, gsem, wsem):
        wid = lax.axis_index("s") * NC + lax.axis_index("c")
        pltpu.sync_copy(idx_hbm.at[wid], idx_v)
        bbase = wid * CH

        def fire_gather(h, half, b):
            pltpu.async_copy(
                table_hbm.at[idx_v.at[h].at[pl.ds(half * HCH, HCH)]],
                rows_v.at[b], gsem.at[b])

        def wait_gather(b):
            pltpu.make_async_copy(
                table_hbm.at[idx_v.at[0].at[pl.ds(0, HCH)]], rows_v.at[b],
                gsem.at[b]).wait()

        def fire_write(h, half, b):
            pltpu.async_copy(
                rows_v.at[b],
                out_hbm.at[h].at[pl.ds(bbase + half * HCH, HCH)], wsem.at[b])

        def wait_write(b):
            pltpu.make_async_copy(rows_v.at[b],
                                  out_hbm.at[0].at[pl.ds(bbase, HCH)],
                                  wsem.at[b]).wait()

        hb = NBUF // 2

        for j in range(NBUF):
            fire_gather(j // 2, j % 2, j)

        @pl.loop(0, hist - hb, step=hb)
        def _(h0):
            for j in range(NBUF):
                wait_gather(j)
                fire_write(h0 + j // 2, j % 2, j)
            for j in range(NBUF):
                wait_write(j)
                fire_gather(h0 + hb + j // 2, j % 2, j)

        for j in range(NBUF):
            wait_gather(j)
            fire_write(hist - hb + j // 2, j % 2, j)
        for j in range(NBUF):
            wait_write(j)

    return body(table, idx3)


def kernel(input_ids, word_embeddings):
    batch, hist = input_ids.shape
    idx3 = input_ids.astype(jnp.int32).reshape(NW, CH, hist)
    idx3 = idx3.transpose(0, 2, 1)
    out = _lookup(idx3, word_embeddings, batch=batch, hist=hist)
    return out.transpose(1, 0, 2)
